# baseline (speedup 1.0000x reference)
import functools

import jax
import jax.numpy as jnp
from jax import lax
from jax.experimental import pallas as pl
from jax.experimental.pallas import tpu as pltpu
from jax.experimental.pallas import tpu_sc as plsc

B, T, D = 4, 2048, 2048
E, C = 8, 512
OUT = 16384
O_E = OUT // E
N_ROWS_E = B * C

NC, NS = 2, 16
NW = NC * NS
ROWS_PER_W = N_ROWS_E // NW
CHUNK = 16
N_CHUNKS = ROWS_PER_W // CHUNK


def _sc_gather_e(x2d, idx_e):
    mesh = plsc.VectorSubcoreMesh(core_axis_name="c", subcore_axis_name="s")

    @functools.partial(
        pl.kernel,
        mesh=mesh,
        out_type=jax.ShapeDtypeStruct((N_ROWS_E, D), jnp.float32),
        scratch_types=[
            pltpu.VMEM((ROWS_PER_W,), jnp.int32),
            pltpu.VMEM((CHUNK, D), jnp.float32),
            pltpu.VMEM((CHUNK, D), jnp.float32),
            pltpu.SemaphoreType.DMA,
            pltpu.SemaphoreType.DMA,
            pltpu.SemaphoreType.DMA,
            pltpu.SemaphoreType.DMA,
        ],
    )
    def gather_kernel(x_hbm, idx_hbm, out_hbm, idx_v, buf_a, buf_b, ga, gb, wa, wb):
        wid = lax.axis_index("s") * NC + lax.axis_index("c")
        base = wid * ROWS_PER_W
        pltpu.sync_copy(idx_hbm.at[pl.ds(base, ROWS_PER_W)], idx_v)

        bufs = (buf_a, buf_b)
        gsems = (ga, gb)
        wsems = (wa, wb)

        def gather_chunk(c):
            cp = pltpu.make_async_copy(
                x_hbm.at[idx_v.at[pl.ds(c * CHUNK, CHUNK)]], bufs[c % 2],
                gsems[c % 2],
            )
            cp.start()
            return cp

        def write_chunk(c):
            cp = pltpu.make_async_copy(
                bufs[c % 2], out_hbm.at[pl.ds(base + c * CHUNK, CHUNK)],
                wsems[c % 2],
            )
            cp.start()
            return cp

        g = [None] * N_CHUNKS
        w = [None] * N_CHUNKS
        g[0] = gather_chunk(0)
        g[1] = gather_chunk(1)
        g[0].wait()
        w[0] = write_chunk(0)
        for c in range(2, N_CHUNKS):
            w[c - 2].wait()
            g[c] = gather_chunk(c)
            g[c - 1].wait()
            w[c - 1] = write_chunk(c - 1)
        g[N_CHUNKS - 1].wait()
        w[N_CHUNKS - 1] = write_chunk(N_CHUNKS - 1)
        w[N_CHUNKS - 2].wait()
        w[N_CHUNKS - 1].wait()

    return gather_kernel(x2d, idx_e)


def _mm_e(e, g_e, We, be, carry):
    def mm_kernel(a_ref, w_ref, b_ref, carry_ref, o_ref):
        del carry_ref
        acc = lax.dot_general(
            a_ref[0], w_ref[0], (((1,), (1,)), ((), ())),
            preferred_element_type=jnp.float32,
        )
        o_ref[0, 0] = acc + b_ref[0]

    return pl.pallas_call(
        mm_kernel,
        grid=(B,),
        in_specs=[
            pl.BlockSpec((1, C, D), lambda b: (b, 0, 0)),
            pl.BlockSpec((1, O_E, D), lambda b: (e, 0, 0)),
            pl.BlockSpec((1, 1, O_E), lambda b: (e, 0, 0)),
            pl.BlockSpec(memory_space=pl.ANY),
        ],
        out_specs=pl.BlockSpec((1, 1, C, O_E), lambda b: (b, e, 0, 0)),
        out_shape=jax.ShapeDtypeStruct((B, E, C, O_E), jnp.float32),
        input_output_aliases={3: 0},
    )(g_e, We, be, carry)


def kernel(x, expert_indices, W, b):
    x2d = x.reshape(B * T, D)
    idx_ebc = jnp.transpose(expert_indices, (1, 0, 2))
    flat_idx = (
        idx_ebc + (jnp.arange(B, dtype=jnp.int32) * T)[None, :, None]
    ).reshape(E, N_ROWS_E)
    We = W.reshape(E, O_E, D)
    be = b.reshape(E, 1, O_E)

    gathered = [
        _sc_gather_e(x2d, flat_idx[e]).reshape(B, C, D) for e in range(E)
    ]
    out = jnp.empty((B, E, C, O_E), dtype=jnp.float32)
    for e in range(E):
        out = _mm_e(e, gathered[e], We, be, out)
    return out


# speedup vs baseline: 1.1948x; 1.1948x over previous
import functools

import jax
import jax.numpy as jnp
from jax import lax
from jax.experimental import pallas as pl
from jax.experimental.pallas import tpu as pltpu
from jax.experimental.pallas import tpu_sc as plsc

B, T, D = 4, 2048, 2048
E, C = 8, 512
OUT = 16384
O_E = OUT // E

N_HALF = 2
E_H = E // N_HALF
N_ROWS_H = E_H * B * C

NC, NS = 2, 16
NW = NC * NS
ROWS_PER_W = N_ROWS_H // NW
CHUNK = 16
N_CHUNKS = ROWS_PER_W // CHUNK


def _sc_gather_half(x2d, idx_h):
    mesh = plsc.VectorSubcoreMesh(core_axis_name="c", subcore_axis_name="s")

    @functools.partial(
        pl.kernel,
        mesh=mesh,
        out_type=jax.ShapeDtypeStruct((N_ROWS_H, D), jnp.float32),
        scratch_types=[
            pltpu.VMEM((ROWS_PER_W,), jnp.int32),
            pltpu.VMEM((CHUNK, D), jnp.float32),
            pltpu.VMEM((CHUNK, D), jnp.float32),
            pltpu.SemaphoreType.DMA,
            pltpu.SemaphoreType.DMA,
            pltpu.SemaphoreType.DMA,
            pltpu.SemaphoreType.DMA,
        ],
    )
    def gather_kernel(x_hbm, idx_hbm, out_hbm, idx_v, buf_a, buf_b, ga, gb, wa, wb):
        wid = lax.axis_index("s") * NC + lax.axis_index("c")
        base = wid * ROWS_PER_W
        pltpu.sync_copy(idx_hbm.at[pl.ds(base, ROWS_PER_W)], idx_v)

        bufs = (buf_a, buf_b)
        gsems = (ga, gb)
        wsems = (wa, wb)

        def gather_chunk(c):
            cp = pltpu.make_async_copy(
                x_hbm.at[idx_v.at[pl.ds(c * CHUNK, CHUNK)]], bufs[c % 2],
                gsems[c % 2],
            )
            cp.start()
            return cp

        def write_chunk(c):
            cp = pltpu.make_async_copy(
                bufs[c % 2], out_hbm.at[pl.ds(base + c * CHUNK, CHUNK)],
                wsems[c % 2],
            )
            cp.start()
            return cp

        g = [None] * N_CHUNKS
        w = [None] * N_CHUNKS
        g[0] = gather_chunk(0)
        g[1] = gather_chunk(1)
        g[0].wait()
        w[0] = write_chunk(0)
        for c in range(2, N_CHUNKS):
            w[c - 2].wait()
            g[c] = gather_chunk(c)
            g[c - 1].wait()
            w[c - 1] = write_chunk(c - 1)
        g[N_CHUNKS - 1].wait()
        w[N_CHUNKS - 1] = write_chunk(N_CHUNKS - 1)
        w[N_CHUNKS - 2].wait()
        w[N_CHUNKS - 1].wait()

    return gather_kernel(x2d, idx_h)


def _mm_half(h, g_h, We, be, carry):
    def mm_kernel(a_ref, w_ref, b_ref, *rest):
        o_ref = rest[-1]
        acc = lax.dot_general(
            a_ref[0], w_ref[0], (((1,), (1,)), ((), ())),
            preferred_element_type=jnp.float32,
        )
        o_ref[0, 0] = acc + b_ref[0]

    in_specs = [
        pl.BlockSpec((1, C, D), lambda e, b: (e * B + b, 0, 0)),
        pl.BlockSpec((1, O_E, D), lambda e, b: (h * E_H + e, 0, 0)),
        pl.BlockSpec((1, 1, O_E), lambda e, b: (h * E_H + e, 0, 0)),
    ]
    args = (g_h, We, be)
    aliases = {}
    if carry is not None:
        in_specs.append(pl.BlockSpec(memory_space=pl.ANY))
        args = args + (carry,)
        aliases = {3: 0}
    return pl.pallas_call(
        mm_kernel,
        grid=(E_H, B),
        in_specs=in_specs,
        out_specs=pl.BlockSpec(
            (1, 1, C, O_E), lambda e, b: (b, h * E_H + e, 0, 0)
        ),
        out_shape=jax.ShapeDtypeStruct((B, E, C, O_E), jnp.float32),
        input_output_aliases=aliases,
    )(*args)


def kernel(x, expert_indices, W, b):
    x2d = x.reshape(B * T, D)
    idx_ebc = jnp.transpose(expert_indices, (1, 0, 2))
    flat_idx = (
        idx_ebc + (jnp.arange(B, dtype=jnp.int32) * T)[None, :, None]
    ).reshape(N_HALF, N_ROWS_H)
    We = W.reshape(E, O_E, D)
    be = b.reshape(E, 1, O_E)

    gathered = [
        _sc_gather_half(x2d, flat_idx[h]).reshape(E_H * B, C, D)
        for h in range(N_HALF)
    ]
    out = None
    for h in range(N_HALF):
        out = _mm_half(h, gathered[h], We, be, out)
    return out
